# baseline (speedup 1.0000x reference)
import functools

import jax
import jax.numpy as jnp
from jax import lax
from jax.experimental import pallas as pl
from jax.experimental.pallas import tpu as pltpu
from jax.experimental.pallas import tpu_sc as plsc

VOCAB = 400002
EMBED = 50
EMBED_P = 56
BATCH = 4096
SEQ = 200

NC, NS = 2, 16
NW = NC * NS
ROWS_PER_W = BATCH // NW
BPC = 4
N_CHUNKS = ROWS_PER_W // BPC
SPLIT = (96, 104)

_mesh = plsc.VectorSubcoreMesh(
    core_axis_name="c", subcore_axis_name="s", num_cores=NC, num_subcores=NS
)


@functools.partial(
    pl.kernel,
    out_type=jax.ShapeDtypeStruct((BATCH, SEQ, EMBED_P), jnp.float32),
    mesh=_mesh,
    scratch_types=[
        pltpu.VMEM((BPC, SEQ), jnp.int32),
        pltpu.VMEM((BPC, SEQ, EMBED_P), jnp.float32),
        pltpu.SemaphoreType.DMA,
    ],
    compiler_params=pltpu.CompilerParams(use_tc_tiling_on_sc=False),
)
def _gather_kernel(table_hbm, idx_hbm, out_hbm, idx_v, rows_v, sem):
    wid = lax.axis_index("s") * NC + lax.axis_index("c")
    brow0 = wid * ROWS_PER_W

    @pl.loop(0, N_CHUNKS)
    def _chunk(m):
        base = brow0 + m * BPC
        pltpu.sync_copy(idx_hbm.at[pl.ds(base, BPC)], idx_v)
        copies = []
        for j in range(BPC):
            off = 0
            for w in SPLIT:
                copies.append(
                    pltpu.async_copy(
                        table_hbm.at[idx_v.at[j, pl.ds(off, w)]],
                        rows_v.at[j, pl.ds(off, w)],
                        sem,
                    )
                )
                off += w
        for c in copies:
            c.wait()
        pltpu.sync_copy(rows_v, out_hbm.at[pl.ds(base, BPC)])


def kernel(input_ids, word_embeddings):
    table_p = jnp.pad(word_embeddings, ((0, 0), (0, EMBED_P - EMBED)))
    out = _gather_kernel(table_p, input_ids.astype(jnp.int32))
    return out[..., :EMBED]
